# baseline (speedup 1.0000x reference)
import dataclasses

import jax
import jax.numpy as jnp
from jax import lax
from jax.experimental import pallas as pl
from jax.experimental.pallas import tpu as pltpu
from jax.experimental.pallas import tpu_sc as plsc

N_USERS = 1000000
N_HORIZONS = 16
BATCH = 16384

NUM_CORES = 2
NUM_SUBCORES = 16
NUM_WORKERS = NUM_CORES * NUM_SUBCORES
ROWS_PER_WORKER = BATCH // NUM_WORKERS
LANES = 16
CHUNKS = ROWS_PER_WORKER // LANES
TAB_ROWS = N_USERS // 8
OUT_ROWS = BATCH * N_HORIZONS // 128
OUT_ROWS_PER_WORKER = OUT_ROWS // NUM_WORKERS


def _extract(src128, out64, idx_v, k):
    iota = lax.iota(jnp.int32, LANES)
    u16 = idx_v[pl.ds(k * LANES, LANES)]
    base16 = (u16 & 7) * LANES
    rows16 = iota + k * LANES
    srow16 = rows16 >> 3
    scol_base16 = (rows16 & 7) * LANES
    for j in range(LANES):
        v = plsc.load_gather(src128, [rows16, base16 + j])
        plsc.store_scatter(out64, [srow16, scol_base16 + j], v)


def _sc_gather_kernel(scale_hbm, bias_hbm, idx_hbm, scale_out, bias_out,
                      idx_v, rows_v, g128_v, outs_v, outb_v, sem):
    wid = lax.axis_index("s") * NUM_CORES + lax.axis_index("c")
    base = wid * ROWS_PER_WORKER
    pltpu.sync_copy(idx_hbm.at[pl.ds(base, ROWS_PER_WORKER)], idx_v)

    @pl.loop(0, CHUNKS)
    def _(k):
        rows_v[pl.ds(k * LANES, LANES)] = idx_v[pl.ds(k * LANES, LANES)] >> 3

    pltpu.async_copy(scale_hbm.at[rows_v], g128_v, sem).wait()

    @pl.loop(0, CHUNKS)
    def _(k):
        _extract(g128_v, outs_v, idx_v, k)

    pltpu.async_copy(bias_hbm.at[rows_v], g128_v, sem).wait()

    @pl.loop(0, CHUNKS)
    def _(k):
        _extract(g128_v, outb_v, idx_v, k)

    obase = wid * OUT_ROWS_PER_WORKER
    pltpu.sync_copy(outs_v, scale_out.at[pl.ds(obase, OUT_ROWS_PER_WORKER)])
    pltpu.sync_copy(outb_v, bias_out.at[pl.ds(obase, OUT_ROWS_PER_WORKER)])


def _sc_gather(scale128, bias128, idx):
    mesh = plsc.VectorSubcoreMesh(core_axis_name="c", subcore_axis_name="s")
    out = jax.ShapeDtypeStruct((OUT_ROWS, 128), jnp.float32)
    kern = pl.kernel(
        _sc_gather_kernel,
        mesh=mesh,
        out_type=(out, out),
        scratch_types=[
            pltpu.VMEM((ROWS_PER_WORKER,), jnp.int32),
            pltpu.VMEM((ROWS_PER_WORKER,), jnp.int32),
            pltpu.VMEM((ROWS_PER_WORKER, 128), jnp.float32),
            pltpu.VMEM((OUT_ROWS_PER_WORKER, 128), jnp.float32),
            pltpu.VMEM((OUT_ROWS_PER_WORKER, 128), jnp.float32),
            pltpu.SemaphoreType.DMA,
        ],
        compiler_params=_sc_compiler_params(),
    )
    return kern(scale128, bias128, idx)


def _sc_compiler_params():
    cp = pltpu.CompilerParams()
    if "needs_layout_passes" in pltpu.CompilerParams.__dataclass_fields__:
        cp = dataclasses.replace(cp, needs_layout_passes=False)
    return cp


def _tc_math_kernel(p_ref, s_ref, b_ref, o_ref):
    eps = 1e-07
    p = jnp.clip(p_ref[...], eps, 1.0 - eps)
    logits = jnp.log(p / (1.0 - p))
    o_ref[...] = jax.nn.sigmoid(logits * s_ref[...] + b_ref[...])


def _tc_math(p2, s2, b2):
    return pl.pallas_call(
        _tc_math_kernel,
        out_shape=jax.ShapeDtypeStruct(p2.shape, jnp.float32),
    )(p2, s2, b2)


@jax.jit
def kernel(probs, user_ids, scale_table, bias_table):
    idx = user_ids.astype(jnp.int32)
    scale_g, bias_g = _sc_gather(scale_table.reshape(TAB_ROWS, 128),
                                 bias_table.reshape(TAB_ROWS, 128), idx)
    out = _tc_math(probs.reshape(OUT_ROWS, 128), scale_g, bias_g)
    return out.reshape(BATCH, N_HORIZONS)
